# Initial kernel scaffold; baseline (speedup 1.0000x reference)
import jax
import jax.numpy as jnp
from jax.experimental import pallas as pl


def kernel(data_batch_1, data_batch_2, edge_index_1, edge_index_2, edge_type_1, edge_type_2, W0, W_rest, rel_emb, attn_l, attn_r, attn_e, conv_bias, ln_gamma, ln_beta, ln1_gamma, ln1_beta, lin1_W, lin1_b):
    raise NotImplementedError("write your pallas kernel here")


# speedup vs baseline: 5.0169x; 5.0169x over previous
import functools

import jax
import jax.numpy as jnp
from jax import lax
from jax.experimental import pallas as pl
from jax.experimental.pallas import tpu as pltpu
from jax.experimental.pallas import tpu_sc as plsc

N = 10000
E = 320000
D = 128
HC = 64
HQ = HC // 4
L = 4
R = 15
HC2 = 256

NC = 2
NS = 16
EPT = E // NS
CH = 80
NCHUNK = EPT // CH

f32 = jnp.float32
i32 = jnp.int32


def _pre_body(pos_ref, feat_ref, g_ref, b_ref, posn_ref, cov_ref, featn_ref):
    pos = pos_ref[...]
    pc = pos - jnp.mean(pos, axis=0, keepdims=True)
    posn = pc * ((1.0 / jnp.max(jnp.abs(pc))) * 0.999999)
    posn_ref[...] = posn
    p = posn - jnp.mean(posn, axis=0, keepdims=True)
    cov_ref[...] = jnp.sum(p[:, :, None] * p[:, None, :], axis=0)
    f = feat_ref[...]
    fm = jnp.mean(f)
    fv = jnp.mean((f - fm) ** 2)
    featn_ref[...] = (f - fm) / jnp.sqrt(fv + 1e-5) * g_ref[...] + b_ref[...]


_pre = pl.pallas_call(
    _pre_body,
    out_shape=(
        jax.ShapeDtypeStruct((N, 3), f32),
        jax.ShapeDtypeStruct((3, 3), f32),
        jax.ShapeDtypeStruct((N, D - 3), f32),
    ),
)


def _d0_body(posn_ref, v_ref, featn_ref, w_ref, al_ref, ar_ref, rel_ref,
             ae_ref, xpe_ref, de_ref):
    x = jnp.concatenate(
        [jnp.dot(posn_ref[...], v_ref[...], preferred_element_type=f32),
         featn_ref[...]], axis=1)
    xp = jnp.dot(x, w_ref[...], preferred_element_type=f32)
    dl = jnp.dot(xp, al_ref[...], preferred_element_type=f32)
    dr = jnp.dot(xp, ar_ref[...], preferred_element_type=f32)
    xpe_ref[...] = jnp.concatenate([xp, dl, dr], axis=1)
    de_ref[...] = jnp.sum(rel_ref[...] * ae_ref[...], axis=1, keepdims=True)


_d0 = pl.pallas_call(
    _d0_body,
    out_shape=(
        jax.ShapeDtypeStruct((N, HC + 2), f32),
        jax.ShapeDtypeStruct((16, 1), f32),
    ),
)


def _mid_body(acc_ref, den_ref, prev_ref, bias_ref, g_ref, b_ref, w_ref,
              al_ref, ar_ref, rel_ref, ae_ref,
              out_ref, xpe_ref, de_ref):
    acc = acc_ref[...]
    den = jnp.sum(den_ref[...], axis=0)
    out = acc / (den[:, None] + 1e-16) + bias_ref[...] + prev_ref[...]
    m = jnp.mean(out)
    v = jnp.mean((out - m) ** 2)
    out = (out - m) / jnp.sqrt(v + 1e-5) * g_ref[...] + b_ref[...]
    out = out * (1.0 / (1.0 + jnp.exp(-out)))
    out_ref[...] = out
    xp = jnp.dot(out, w_ref[...], preferred_element_type=f32)
    dl = jnp.dot(xp, al_ref[...], preferred_element_type=f32)
    dr = jnp.dot(xp, ar_ref[...], preferred_element_type=f32)
    xpe_ref[...] = jnp.concatenate([xp, dl, dr], axis=1)
    de_ref[...] = jnp.sum(rel_ref[...] * ae_ref[...], axis=1, keepdims=True)


_mid = pl.pallas_call(
    _mid_body,
    out_shape=(
        jax.ShapeDtypeStruct((N, HC), f32),
        jax.ShapeDtypeStruct((N, HC + 2), f32),
        jax.ShapeDtypeStruct((16, 1), f32),
    ),
)


def _fin_body(acc_ref, den_ref, prev_ref, bias_ref, g_ref, b_ref, w1_ref,
              b1_ref, out_ref):
    acc = acc_ref[...]
    den = jnp.sum(den_ref[...], axis=0)
    out = acc / (den[:, None] + 1e-16) + bias_ref[...] + prev_ref[...]
    m = jnp.mean(out)
    v = jnp.mean((out - m) ** 2)
    out = (out - m) / jnp.sqrt(v + 1e-5) * g_ref[...] + b_ref[...]
    out = out * (1.0 / (1.0 + jnp.exp(-out)))
    x3 = jnp.mean(out, axis=0, keepdims=True)
    xo = jnp.dot(x3, w1_ref[...], preferred_element_type=f32) + b1_ref[...]
    out_ref[...] = xo * (1.0 / (1.0 + jnp.exp(-xo)))


_fin = pl.pallas_call(
    _fin_body,
    out_shape=jax.ShapeDtypeStruct((1, HC2), f32),
)


def _sc_edge_body(xpf_hbm, dl_hbm, dr_hbm, de_hbm, src_hbm, dst_hbm, et_hbm,
                  zacc_hbm, zden_hbm, acc_out, den_out,
                  dl_v, dr_v, de_v, srcv, dstv, etv, exv, denv,
                  schunk, sidx, dchunk, rows_v, acc_sh, sem):
    c = lax.axis_index("c")
    s = lax.axis_index("s")
    base = s * EPT

    pltpu.sync_copy(dl_hbm, dl_v)
    pltpu.sync_copy(dr_hbm, dr_v)
    pltpu.sync_copy(de_hbm, de_v)
    pltpu.sync_copy(src_hbm.at[pl.ds(base, EPT)], srcv)
    pltpu.sync_copy(dst_hbm.at[pl.ds(base, EPT)], dstv)
    pltpu.sync_copy(et_hbm.at[pl.ds(base, EPT)], etv)
    pltpu.sync_copy(zden_hbm, denv)

    def p1(i, carry):
        dv = dstv[pl.ds(i * 16, 16)]
        sv = srcv[pl.ds(i * 16, 16)]
        tv = etv[pl.ds(i * 16, 16)]
        a = (plsc.load_gather(dl_v, [dv]) + plsc.load_gather(dr_v, [sv])
             + plsc.load_gather(de_v, [tv]))
        ex = jnp.exp(a * (1.0 / (1.0 + jnp.exp(-a))))
        exv[pl.ds(i * 16, 16)] = ex
        plsc.addupdate_scatter(denv, [dv], ex)
        return carry

    lax.fori_loop(0, EPT // 16, p1, 0)

    @pl.when(c == 0)
    def _emit_den():
        pltpu.sync_copy(denv, den_out.at[s])

    for h in range(2):
        q = 2 * c + h
        coff = q * N

        @pl.when(s == 0)
        def _zero_acc():
            pltpu.sync_copy(zacc_hbm, acc_sh)

        plsc.subcore_barrier()

        def p2(j, carry):
            off = base + j * CH
            pltpu.sync_copy(src_hbm.at[pl.ds(off, CH)], schunk)
            pltpu.sync_copy(dst_hbm.at[pl.ds(off, CH)], dchunk)

            def mkidx(r, c2):
                sidx[pl.ds(r * 16, 16)] = schunk[pl.ds(r * 16, 16)] + coff
                return c2

            lax.fori_loop(0, CH // 16, mkidx, 0)
            pltpu.async_copy(xpf_hbm.at[sidx], rows_v, sem).wait()

            def scale(r, c2):
                e = plsc.load_gather(exv, [jnp.full((16,), j * CH, i32) + r])
                rows_v[r, pl.ds(0, 16)] = rows_v[r, pl.ds(0, 16)] * e
                return c2

            lax.fori_loop(0, CH, scale, 0)
            pltpu.sync_copy(rows_v, acc_sh.at[dchunk], add=True)
            return carry

        lax.fori_loop(0, NCHUNK, p2, 0)
        plsc.subcore_barrier()

        @pl.when(s == 0)
        def _emit_acc():
            pltpu.sync_copy(acc_sh, acc_out.at[q])


_sc_edge = functools.partial(
    pl.kernel,
    out_type=[
        jax.ShapeDtypeStruct((4, N, HQ), f32),
        jax.ShapeDtypeStruct((NS, N), f32),
    ],
    mesh=plsc.VectorSubcoreMesh(core_axis_name="c", subcore_axis_name="s"),
    compiler_params=pltpu.CompilerParams(needs_layout_passes=False,
                                         use_tc_tiling_on_sc=False),
    scratch_types=[
        pltpu.VMEM((N,), f32),
        pltpu.VMEM((N,), f32),
        pltpu.VMEM((16,), f32),
        pltpu.VMEM((EPT,), i32),
        pltpu.VMEM((EPT,), i32),
        pltpu.VMEM((EPT,), i32),
        pltpu.VMEM((EPT,), f32),
        pltpu.VMEM((N,), f32),
        pltpu.VMEM((CH,), i32),
        pltpu.VMEM((CH,), i32),
        pltpu.VMEM((CH,), i32),
        pltpu.VMEM((CH, HQ), f32),
        pltpu.VMEM_SHARED((N, HQ), f32),
        pltpu.SemaphoreType.DMA,
    ],
)(_sc_edge_body)


def kernel(data_batch_1, data_batch_2, edge_index_1, edge_index_2,
           edge_type_1, edge_type_2, W0, W_rest, rel_emb,
           attn_l, attn_r, attn_e, conv_bias, ln_gamma, ln_beta,
           ln1_gamma, ln1_beta, lin1_W, lin1_b):
    pos = data_batch_1[:, :3]
    feat = data_batch_1[:, 3:]
    src = edge_index_1[0]
    dst = edge_index_1[1]
    et = edge_type_1

    posn, cov, featn = _pre(pos, feat, ln1_gamma.reshape(1, D - 3),
                            ln1_beta.reshape(1, D - 3))
    ev, V = jnp.linalg.eigh(cov)
    V = V[:, jnp.argsort(-ev)]

    rel_pad = jnp.concatenate([rel_emb, jnp.zeros((L, 1, HC), f32)], axis=1)
    zacc = jnp.zeros((N, HQ), f32)
    zden = jnp.zeros((N,), f32)

    xpe, de = _d0(posn, V, featn, W0,
                  attn_l[0].reshape(HC, 1), attn_r[0].reshape(HC, 1),
                  rel_pad[0], attn_e[0].reshape(1, HC))

    prev = jnp.zeros((N, HC), f32)
    for i in range(L):
        xpf = xpe[:, :HC].reshape(N, 4, HQ).transpose(1, 0, 2).reshape(4 * N, HQ)
        acc4, den16 = _sc_edge(xpf, xpe[:, HC], xpe[:, HC + 1],
                               de.reshape(16), src, dst, et, zacc, zden)
        acc = acc4.transpose(1, 0, 2).reshape(N, HC)
        if i < L - 1:
            prev, xpe, de = _mid(
                acc, den16, prev, conv_bias[i].reshape(1, HC),
                ln_gamma[i].reshape(1, HC), ln_beta[i].reshape(1, HC),
                W_rest[i], attn_l[i + 1].reshape(HC, 1),
                attn_r[i + 1].reshape(HC, 1), rel_pad[i + 1],
                attn_e[i + 1].reshape(1, HC))
        else:
            xo = _fin(acc, den16, prev, conv_bias[i].reshape(1, HC),
                      ln_gamma[i].reshape(1, HC), ln_beta[i].reshape(1, HC),
                      lin1_W, lin1_b.reshape(1, HC2))
    return (xo, xo)


# speedup vs baseline: 9.4745x; 1.8885x over previous
import functools

import jax
import jax.numpy as jnp
from jax import lax
from jax.experimental import pallas as pl
from jax.experimental.pallas import tpu as pltpu
from jax.experimental.pallas import tpu_sc as plsc

N = 10000
E = 320000
D = 128
HC = 64
HQ = HC // 4
L = 4
R = 15
HC2 = 256

NC = 2
NS = 16
EPT = E // NS
CH = 400
NCHUNK = EPT // CH

f32 = jnp.float32
i32 = jnp.int32


def _pre_body(pos_ref, feat_ref, g_ref, b_ref, posn_ref, cov_ref, featn_ref):
    pos = pos_ref[...]
    pc = pos - jnp.mean(pos, axis=0, keepdims=True)
    posn = pc * ((1.0 / jnp.max(jnp.abs(pc))) * 0.999999)
    posn_ref[...] = posn
    p = posn - jnp.mean(posn, axis=0, keepdims=True)
    cov_ref[...] = jnp.sum(p[:, :, None] * p[:, None, :], axis=0)
    f = feat_ref[...]
    fm = jnp.mean(f)
    fv = jnp.mean((f - fm) ** 2)
    featn_ref[...] = (f - fm) / jnp.sqrt(fv + 1e-5) * g_ref[...] + b_ref[...]


_pre = pl.pallas_call(
    _pre_body,
    out_shape=(
        jax.ShapeDtypeStruct((N, 3), f32),
        jax.ShapeDtypeStruct((3, 3), f32),
        jax.ShapeDtypeStruct((N, D - 3), f32),
    ),
)


def _d0_body(posn_ref, v_ref, featn_ref, w_ref, al_ref, ar_ref, rel_ref,
             ae_ref, xpe_ref, de_ref):
    x = jnp.concatenate(
        [jnp.dot(posn_ref[...], v_ref[...], preferred_element_type=f32),
         featn_ref[...]], axis=1)
    xp = jnp.dot(x, w_ref[...], preferred_element_type=f32)
    dl = jnp.dot(xp, al_ref[...], preferred_element_type=f32)
    dr = jnp.dot(xp, ar_ref[...], preferred_element_type=f32)
    xpe_ref[...] = jnp.concatenate([xp, dl, dr], axis=1)
    de_ref[...] = jnp.sum(rel_ref[...] * ae_ref[...], axis=1, keepdims=True)


_d0 = pl.pallas_call(
    _d0_body,
    out_shape=(
        jax.ShapeDtypeStruct((N, HC + 2), f32),
        jax.ShapeDtypeStruct((16, 1), f32),
    ),
)


def _mid_body(acc_ref, den_ref, prev_ref, bias_ref, g_ref, b_ref, w_ref,
              al_ref, ar_ref, rel_ref, ae_ref,
              out_ref, xpe_ref, de_ref):
    acc = acc_ref[...]
    den = jnp.sum(den_ref[...], axis=0)
    out = acc / (den[:, None] + 1e-16) + bias_ref[...] + prev_ref[...]
    m = jnp.mean(out)
    v = jnp.mean((out - m) ** 2)
    out = (out - m) / jnp.sqrt(v + 1e-5) * g_ref[...] + b_ref[...]
    out = out * (1.0 / (1.0 + jnp.exp(-out)))
    out_ref[...] = out
    xp = jnp.dot(out, w_ref[...], preferred_element_type=f32)
    dl = jnp.dot(xp, al_ref[...], preferred_element_type=f32)
    dr = jnp.dot(xp, ar_ref[...], preferred_element_type=f32)
    xpe_ref[...] = jnp.concatenate([xp, dl, dr], axis=1)
    de_ref[...] = jnp.sum(rel_ref[...] * ae_ref[...], axis=1, keepdims=True)


_mid = pl.pallas_call(
    _mid_body,
    out_shape=(
        jax.ShapeDtypeStruct((N, HC), f32),
        jax.ShapeDtypeStruct((N, HC + 2), f32),
        jax.ShapeDtypeStruct((16, 1), f32),
    ),
)


def _fin_body(acc_ref, den_ref, prev_ref, bias_ref, g_ref, b_ref, w1_ref,
              b1_ref, out_ref):
    acc = acc_ref[...]
    den = jnp.sum(den_ref[...], axis=0)
    out = acc / (den[:, None] + 1e-16) + bias_ref[...] + prev_ref[...]
    m = jnp.mean(out)
    v = jnp.mean((out - m) ** 2)
    out = (out - m) / jnp.sqrt(v + 1e-5) * g_ref[...] + b_ref[...]
    out = out * (1.0 / (1.0 + jnp.exp(-out)))
    x3 = jnp.mean(out, axis=0, keepdims=True)
    xo = jnp.dot(x3, w1_ref[...], preferred_element_type=f32) + b1_ref[...]
    out_ref[...] = xo * (1.0 / (1.0 + jnp.exp(-xo)))


_fin = pl.pallas_call(
    _fin_body,
    out_shape=jax.ShapeDtypeStruct((1, HC2), f32),
)


def _sc_edge_body(xpf_hbm, dl_hbm, dr_hbm, de_hbm, src_hbm, dst_hbm, et_hbm,
                  zacc_hbm, zden_hbm, acc_out, den_out,
                  dl_v, dr_v, de_v, srcv, dstv, etv, exv, denv,
                  schunk, sidx, dchunk, rows_v, acc_sh, sem):
    c = lax.axis_index("c")
    s = lax.axis_index("s")
    base = s * EPT

    pltpu.sync_copy(dl_hbm, dl_v)
    pltpu.sync_copy(dr_hbm, dr_v)
    pltpu.sync_copy(de_hbm, de_v)
    pltpu.sync_copy(src_hbm.at[pl.ds(base, EPT)], srcv)
    pltpu.sync_copy(dst_hbm.at[pl.ds(base, EPT)], dstv)
    pltpu.sync_copy(et_hbm.at[pl.ds(base, EPT)], etv)
    pltpu.sync_copy(zden_hbm, denv)

    def p1(i, carry):
        dv = dstv[pl.ds(i * 16, 16)]
        sv = srcv[pl.ds(i * 16, 16)]
        tv = etv[pl.ds(i * 16, 16)]
        a = (plsc.load_gather(dl_v, [dv]) + plsc.load_gather(dr_v, [sv])
             + plsc.load_gather(de_v, [tv]))
        ex = jnp.exp(a * (1.0 / (1.0 + jnp.exp(-a))))
        exv[pl.ds(i * 16, 16)] = ex
        plsc.addupdate_scatter(denv, [dv], ex)
        return carry

    lax.fori_loop(0, EPT // 16, p1, 0)

    @pl.when(c == 0)
    def _emit_den():
        pltpu.sync_copy(denv, den_out.at[s])

    for h in range(2):
        q = 2 * c + h
        coff = q * N

        @pl.when(s == 0)
        def _zero_acc():
            pltpu.sync_copy(zacc_hbm, acc_sh)

        plsc.subcore_barrier()

        def p2(j, carry):
            off = base + j * CH
            pltpu.sync_copy(src_hbm.at[pl.ds(off, CH)], schunk)
            pltpu.sync_copy(dst_hbm.at[pl.ds(off, CH)], dchunk)

            def mkidx(r, c2):
                sidx[pl.ds(r * 16, 16)] = schunk[pl.ds(r * 16, 16)] + coff
                return c2

            lax.fori_loop(0, CH // 16, mkidx, 0)
            pltpu.async_copy(xpf_hbm.at[sidx], rows_v, sem).wait()

            def scale(r, c2):
                e = plsc.load_gather(exv, [jnp.full((16,), j * CH, i32) + r])
                rows_v[r, pl.ds(0, 16)] = rows_v[r, pl.ds(0, 16)] * e
                return c2

            lax.fori_loop(0, CH, scale, 0)
            pltpu.sync_copy(rows_v, acc_sh.at[dchunk], add=True)
            return carry

        lax.fori_loop(0, NCHUNK, p2, 0)
        plsc.subcore_barrier()

        @pl.when(s == 0)
        def _emit_acc():
            pltpu.sync_copy(acc_sh, acc_out.at[q])


_sc_edge = functools.partial(
    pl.kernel,
    out_type=[
        jax.ShapeDtypeStruct((4, N, HQ), f32),
        jax.ShapeDtypeStruct((NS, N), f32),
    ],
    mesh=plsc.VectorSubcoreMesh(core_axis_name="c", subcore_axis_name="s"),
    compiler_params=pltpu.CompilerParams(needs_layout_passes=False,
                                         use_tc_tiling_on_sc=False),
    scratch_types=[
        pltpu.VMEM((N,), f32),
        pltpu.VMEM((N,), f32),
        pltpu.VMEM((16,), f32),
        pltpu.VMEM((EPT,), i32),
        pltpu.VMEM((EPT,), i32),
        pltpu.VMEM((EPT,), i32),
        pltpu.VMEM((EPT,), f32),
        pltpu.VMEM((N,), f32),
        pltpu.VMEM((CH,), i32),
        pltpu.VMEM((CH,), i32),
        pltpu.VMEM((CH,), i32),
        pltpu.VMEM((CH, HQ), f32),
        pltpu.VMEM_SHARED((N, HQ), f32),
        pltpu.SemaphoreType.DMA,
    ],
)(_sc_edge_body)


def kernel(data_batch_1, data_batch_2, edge_index_1, edge_index_2,
           edge_type_1, edge_type_2, W0, W_rest, rel_emb,
           attn_l, attn_r, attn_e, conv_bias, ln_gamma, ln_beta,
           ln1_gamma, ln1_beta, lin1_W, lin1_b):
    pos = data_batch_1[:, :3]
    feat = data_batch_1[:, 3:]
    src = edge_index_1[0]
    dst = edge_index_1[1]
    et = edge_type_1

    posn, cov, featn = _pre(pos, feat, ln1_gamma.reshape(1, D - 3),
                            ln1_beta.reshape(1, D - 3))
    ev, V = jnp.linalg.eigh(cov)
    V = V[:, jnp.argsort(-ev)]

    rel_pad = jnp.concatenate([rel_emb, jnp.zeros((L, 1, HC), f32)], axis=1)
    zacc = jnp.zeros((N, HQ), f32)
    zden = jnp.zeros((N,), f32)

    xpe, de = _d0(posn, V, featn, W0,
                  attn_l[0].reshape(HC, 1), attn_r[0].reshape(HC, 1),
                  rel_pad[0], attn_e[0].reshape(1, HC))

    prev = jnp.zeros((N, HC), f32)
    for i in range(L):
        xpf = xpe[:, :HC].reshape(N, 4, HQ).transpose(1, 0, 2).reshape(4 * N, HQ)
        acc4, den16 = _sc_edge(xpf, xpe[:, HC], xpe[:, HC + 1],
                               de.reshape(16), src, dst, et, zacc, zden)
        acc = acc4.transpose(1, 0, 2).reshape(N, HC)
        if i < L - 1:
            prev, xpe, de = _mid(
                acc, den16, prev, conv_bias[i].reshape(1, HC),
                ln_gamma[i].reshape(1, HC), ln_beta[i].reshape(1, HC),
                W_rest[i], attn_l[i + 1].reshape(HC, 1),
                attn_r[i + 1].reshape(HC, 1), rel_pad[i + 1],
                attn_e[i + 1].reshape(1, HC))
        else:
            xo = _fin(acc, den16, prev, conv_bias[i].reshape(1, HC),
                      ln_gamma[i].reshape(1, HC), ln_beta[i].reshape(1, HC),
                      lin1_W, lin1_b.reshape(1, HC2))
    return (xo, xo)


# speedup vs baseline: 12.8680x; 1.3582x over previous
import functools

import jax
import jax.numpy as jnp
from jax import lax
from jax.experimental import pallas as pl
from jax.experimental.pallas import tpu as pltpu
from jax.experimental.pallas import tpu_sc as plsc

N = 10000
E = 320000
D = 128
HC = 64
HQ = HC // 4
L = 4
R = 15
HC2 = 256

NC = 2
NS = 16
EPT = E // NS
CH = 400
NCHUNK = EPT // CH

f32 = jnp.float32
i32 = jnp.int32


def _pre_body(pos_ref, feat_ref, g_ref, b_ref, posn_ref, cov_ref, featn_ref):
    pos = pos_ref[...]
    pc = pos - jnp.mean(pos, axis=0, keepdims=True)
    posn = pc * ((1.0 / jnp.max(jnp.abs(pc))) * 0.999999)
    posn_ref[...] = posn
    p = posn - jnp.mean(posn, axis=0, keepdims=True)
    cov_ref[...] = jnp.sum(p[:, :, None] * p[:, None, :], axis=0)
    f = feat_ref[...]
    fm = jnp.mean(f)
    fv = jnp.mean((f - fm) ** 2)
    featn_ref[...] = (f - fm) / jnp.sqrt(fv + 1e-5) * g_ref[...] + b_ref[...]


_pre = pl.pallas_call(
    _pre_body,
    out_shape=(
        jax.ShapeDtypeStruct((N, 3), f32),
        jax.ShapeDtypeStruct((3, 3), f32),
        jax.ShapeDtypeStruct((N, D - 3), f32),
    ),
)


def _d0_body(posn_ref, v_ref, featn_ref, w_ref, al_ref, ar_ref, rel_ref,
             ae_ref, xpe_ref, de_ref):
    x = jnp.concatenate(
        [jnp.dot(posn_ref[...], v_ref[...], preferred_element_type=f32),
         featn_ref[...]], axis=1)
    xp = jnp.dot(x, w_ref[...], preferred_element_type=f32)
    dl = jnp.dot(xp, al_ref[...], preferred_element_type=f32)
    dr = jnp.dot(xp, ar_ref[...], preferred_element_type=f32)
    xpe_ref[...] = jnp.concatenate([xp, dl, dr], axis=1)
    de_ref[...] = jnp.sum(rel_ref[...] * ae_ref[...], axis=1, keepdims=True)


_d0 = pl.pallas_call(
    _d0_body,
    out_shape=(
        jax.ShapeDtypeStruct((N, HC + 2), f32),
        jax.ShapeDtypeStruct((16, 1), f32),
    ),
)


def _mid_body(acc_ref, den_ref, prev_ref, bias_ref, g_ref, b_ref, w_ref,
              al_ref, ar_ref, rel_ref, ae_ref,
              out_ref, xpe_ref, de_ref):
    acc = acc_ref[...]
    den = jnp.sum(den_ref[...], axis=0)
    out = acc / (den[:, None] + 1e-16) + bias_ref[...] + prev_ref[...]
    m = jnp.mean(out)
    v = jnp.mean((out - m) ** 2)
    out = (out - m) / jnp.sqrt(v + 1e-5) * g_ref[...] + b_ref[...]
    out = out * (1.0 / (1.0 + jnp.exp(-out)))
    out_ref[...] = out
    xp = jnp.dot(out, w_ref[...], preferred_element_type=f32)
    dl = jnp.dot(xp, al_ref[...], preferred_element_type=f32)
    dr = jnp.dot(xp, ar_ref[...], preferred_element_type=f32)
    xpe_ref[...] = jnp.concatenate([xp, dl, dr], axis=1)
    de_ref[...] = jnp.sum(rel_ref[...] * ae_ref[...], axis=1, keepdims=True)


_mid = pl.pallas_call(
    _mid_body,
    out_shape=(
        jax.ShapeDtypeStruct((N, HC), f32),
        jax.ShapeDtypeStruct((N, HC + 2), f32),
        jax.ShapeDtypeStruct((16, 1), f32),
    ),
)


def _fin_body(acc_ref, den_ref, prev_ref, bias_ref, g_ref, b_ref, w1_ref,
              b1_ref, out_ref):
    acc = acc_ref[...]
    den = jnp.sum(den_ref[...], axis=0)
    out = acc / (den[:, None] + 1e-16) + bias_ref[...] + prev_ref[...]
    m = jnp.mean(out)
    v = jnp.mean((out - m) ** 2)
    out = (out - m) / jnp.sqrt(v + 1e-5) * g_ref[...] + b_ref[...]
    out = out * (1.0 / (1.0 + jnp.exp(-out)))
    x3 = jnp.mean(out, axis=0, keepdims=True)
    xo = jnp.dot(x3, w1_ref[...], preferred_element_type=f32) + b1_ref[...]
    out_ref[...] = xo * (1.0 / (1.0 + jnp.exp(-xo)))


_fin = pl.pallas_call(
    _fin_body,
    out_shape=jax.ShapeDtypeStruct((1, HC2), f32),
)


def _sc_edge_body(xpf_hbm, dl_hbm, dr_hbm, de_hbm, src_hbm, dst_hbm, et_hbm,
                  zacc_hbm, zden_hbm, acc_out, den_out,
                  dl_v, dr_v, de_v, srcv, dstv, exv, denv,
                  sidx0, sidx1, dchunk, rows0, rows1, acc_sh, sem0, sem1):
    c = lax.axis_index("c")
    s = lax.axis_index("s")
    base = s * EPT

    pltpu.sync_copy(dl_hbm, dl_v)
    pltpu.sync_copy(dr_hbm, dr_v)
    pltpu.sync_copy(de_hbm, de_v)
    pltpu.sync_copy(src_hbm.at[pl.ds(base, EPT)], srcv)
    pltpu.sync_copy(dst_hbm.at[pl.ds(base, EPT)], dstv)
    pltpu.sync_copy(zden_hbm, denv)

    def p1c(jc, carry):
        pltpu.sync_copy(et_hbm.at[pl.ds(base + jc * CH, CH)], dchunk)

        def p1(i, c2):
            dv = dstv[pl.ds(jc * CH + i * 16, 16)]
            sv = srcv[pl.ds(jc * CH + i * 16, 16)]
            tv = dchunk[pl.ds(i * 16, 16)]
            a = (plsc.load_gather(dl_v, [dv]) + plsc.load_gather(dr_v, [sv])
                 + plsc.load_gather(de_v, [tv]))
            ex = jnp.exp(a * (1.0 / (1.0 + jnp.exp(-a))))
            exv[pl.ds(jc * CH + i * 16, 16)] = ex
            plsc.addupdate_scatter(denv, [dv], ex)
            return c2

        lax.fori_loop(0, CH // 16, p1, 0)
        return carry

    lax.fori_loop(0, EPT // CH, p1c, 0)

    @pl.when(c == 0)
    def _emit_den():
        pltpu.sync_copy(denv, den_out.at[s])

    def _mkidx(buf, j, add_off):
        def mk(r, c2):
            buf[pl.ds(r * 16, 16)] = srcv[pl.ds(j * CH + r * 16, 16)] + add_off
            return c2

        lax.fori_loop(0, CH // 16, mk, 0)

    def _mkdst(j):
        def mk(r, c2):
            dchunk[pl.ds(r * 16, 16)] = dstv[pl.ds(j * CH + r * 16, 16)]
            return c2

        lax.fori_loop(0, CH // 16, mk, 0)

    sidx = (sidx0, sidx1)
    rows = (rows0, rows1)
    sems = (sem0, sem1)
    for h in range(2):
        q = 2 * c + h
        coff = q * N

        @pl.when(s == 0)
        def _zero_acc():
            pltpu.sync_copy(zacc_hbm, acc_sh)

        plsc.subcore_barrier()

        for b in range(2):
            _mkidx(sidx[b], b, coff)
            pltpu.async_copy(xpf_hbm.at[sidx[b]], rows[b], sems[b])

        def grp(g, carry):
            for b in range(2):
                j = g * 2 + b
                pltpu.make_async_copy(xpf_hbm.at[sidx[b]], rows[b],
                                      sems[b]).wait()

                def scale(r, c2, _b=b, _j=j):
                    e = plsc.load_gather(
                        exv, [jnp.full((16,), _j * CH, i32) + r])
                    rows[_b][r, pl.ds(0, 16)] = rows[_b][r, pl.ds(0, 16)] * e
                    return c2

                lax.fori_loop(0, CH, scale, 0)
                _mkdst(j)
                pltpu.sync_copy(rows[b], acc_sh.at[dchunk], add=True)

                @pl.when(j + 2 < NCHUNK)
                def _prefetch(_b=b, _j=j):
                    _mkidx(sidx[_b], _j + 2, coff)
                    pltpu.async_copy(xpf_hbm.at[sidx[_b]], rows[_b], sems[_b])

            return carry

        lax.fori_loop(0, NCHUNK // 2, grp, 0)
        plsc.subcore_barrier()

        @pl.when(s == 0)
        def _emit_acc():
            pltpu.sync_copy(acc_sh, acc_out.at[q])


_sc_edge = functools.partial(
    pl.kernel,
    out_type=[
        jax.ShapeDtypeStruct((4, N, HQ), f32),
        jax.ShapeDtypeStruct((NS, N), f32),
    ],
    mesh=plsc.VectorSubcoreMesh(core_axis_name="c", subcore_axis_name="s"),
    compiler_params=pltpu.CompilerParams(needs_layout_passes=False,
                                         use_tc_tiling_on_sc=False),
    scratch_types=[
        pltpu.VMEM((N,), f32),
        pltpu.VMEM((N,), f32),
        pltpu.VMEM((16,), f32),
        pltpu.VMEM((EPT,), i32),
        pltpu.VMEM((EPT,), i32),
        pltpu.VMEM((EPT,), f32),
        pltpu.VMEM((N,), f32),
        pltpu.VMEM((CH,), i32),
        pltpu.VMEM((CH,), i32),
        pltpu.VMEM((CH,), i32),
        pltpu.VMEM((CH, HQ), f32),
        pltpu.VMEM((CH, HQ), f32),
        pltpu.VMEM_SHARED((N, HQ), f32),
        pltpu.SemaphoreType.DMA,
        pltpu.SemaphoreType.DMA,
    ],
)(_sc_edge_body)


def kernel(data_batch_1, data_batch_2, edge_index_1, edge_index_2,
           edge_type_1, edge_type_2, W0, W_rest, rel_emb,
           attn_l, attn_r, attn_e, conv_bias, ln_gamma, ln_beta,
           ln1_gamma, ln1_beta, lin1_W, lin1_b):
    pos = data_batch_1[:, :3]
    feat = data_batch_1[:, 3:]
    src = edge_index_1[0]
    dst = edge_index_1[1]
    et = edge_type_1

    posn, cov, featn = _pre(pos, feat, ln1_gamma.reshape(1, D - 3),
                            ln1_beta.reshape(1, D - 3))
    ev, V = jnp.linalg.eigh(cov)
    V = V[:, jnp.argsort(-ev)]

    rel_pad = jnp.concatenate([rel_emb, jnp.zeros((L, 1, HC), f32)], axis=1)
    zacc = jnp.zeros((N, HQ), f32)
    zden = jnp.zeros((N,), f32)

    xpe, de = _d0(posn, V, featn, W0,
                  attn_l[0].reshape(HC, 1), attn_r[0].reshape(HC, 1),
                  rel_pad[0], attn_e[0].reshape(1, HC))

    prev = jnp.zeros((N, HC), f32)
    for i in range(L):
        xpf = xpe[:, :HC].reshape(N, 4, HQ).transpose(1, 0, 2).reshape(4 * N, HQ)
        acc4, den16 = _sc_edge(xpf, xpe[:, HC], xpe[:, HC + 1],
                               de.reshape(16), src, dst, et, zacc, zden)
        acc = acc4.transpose(1, 0, 2).reshape(N, HC)
        if i < L - 1:
            prev, xpe, de = _mid(
                acc, den16, prev, conv_bias[i].reshape(1, HC),
                ln_gamma[i].reshape(1, HC), ln_beta[i].reshape(1, HC),
                W_rest[i], attn_l[i + 1].reshape(HC, 1),
                attn_r[i + 1].reshape(HC, 1), rel_pad[i + 1],
                attn_e[i + 1].reshape(1, HC))
        else:
            xo = _fin(acc, den16, prev, conv_bias[i].reshape(1, HC),
                      ln_gamma[i].reshape(1, HC), ln_beta[i].reshape(1, HC),
                      lin1_W, lin1_b.reshape(1, HC2))
    return (xo, xo)


# speedup vs baseline: 23.4514x; 1.8225x over previous
import functools

import jax
import jax.numpy as jnp
from jax import lax
from jax.experimental import pallas as pl
from jax.experimental.pallas import tpu as pltpu
from jax.experimental.pallas import tpu_sc as plsc

N = 10000
E = 320000
D = 128
HC = 64
HQ = HC // 4
L = 4
R = 15
HC2 = 256

NC = 2
NS = 16
EPT = E // NS
CH = 400
NCHUNK = EPT // CH

f32 = jnp.float32
i32 = jnp.int32


def _pre_body(pos_ref, feat_ref, g_ref, b_ref, posn_ref, cov_ref, featn_ref):
    pos = pos_ref[...]
    pc = pos - jnp.mean(pos, axis=0, keepdims=True)
    posn = pc * ((1.0 / jnp.max(jnp.abs(pc))) * 0.999999)
    posn_ref[...] = posn
    p = posn - jnp.mean(posn, axis=0, keepdims=True)
    cov_ref[...] = jnp.sum(p[:, :, None] * p[:, None, :], axis=0)
    f = feat_ref[...]
    fm = jnp.mean(f)
    fv = jnp.mean((f - fm) ** 2)
    featn_ref[...] = (f - fm) / jnp.sqrt(fv + 1e-5) * g_ref[...] + b_ref[...]


_pre = pl.pallas_call(
    _pre_body,
    out_shape=(
        jax.ShapeDtypeStruct((N, 3), f32),
        jax.ShapeDtypeStruct((3, 3), f32),
        jax.ShapeDtypeStruct((N, D - 3), f32),
    ),
)


def _d0_body(posn_ref, v_ref, featn_ref, w_ref, al_ref, ar_ref, rel_ref,
             ae_ref, xpe_ref, de_ref):
    x = jnp.concatenate(
        [jnp.dot(posn_ref[...], v_ref[...], preferred_element_type=f32),
         featn_ref[...]], axis=1)
    xp = jnp.dot(x, w_ref[...], preferred_element_type=f32)
    dl = jnp.dot(xp, al_ref[...], preferred_element_type=f32)
    dr = jnp.dot(xp, ar_ref[...], preferred_element_type=f32)
    xpe_ref[...] = jnp.concatenate([xp, dl, dr], axis=1)
    de_ref[...] = jnp.sum(rel_ref[...] * ae_ref[...], axis=1, keepdims=True)


_d0 = pl.pallas_call(
    _d0_body,
    out_shape=(
        jax.ShapeDtypeStruct((N, HC + 2), f32),
        jax.ShapeDtypeStruct((16, 1), f32),
    ),
)


def _mid_body(acc_ref, den_ref, prev_ref, bias_ref, g_ref, b_ref, w_ref,
              al_ref, ar_ref, rel_ref, ae_ref,
              out_ref, xpe_ref, de_ref):
    acc = acc_ref[...]
    den = jnp.sum(den_ref[...], axis=0)
    out = acc / (den[:, None] + 1e-16) + bias_ref[...] + prev_ref[...]
    m = jnp.mean(out)
    v = jnp.mean((out - m) ** 2)
    out = (out - m) / jnp.sqrt(v + 1e-5) * g_ref[...] + b_ref[...]
    out = out * (1.0 / (1.0 + jnp.exp(-out)))
    out_ref[...] = out
    xp = jnp.dot(out, w_ref[...], preferred_element_type=f32)
    dl = jnp.dot(xp, al_ref[...], preferred_element_type=f32)
    dr = jnp.dot(xp, ar_ref[...], preferred_element_type=f32)
    xpe_ref[...] = jnp.concatenate([xp, dl, dr], axis=1)
    de_ref[...] = jnp.sum(rel_ref[...] * ae_ref[...], axis=1, keepdims=True)


_mid = pl.pallas_call(
    _mid_body,
    out_shape=(
        jax.ShapeDtypeStruct((N, HC), f32),
        jax.ShapeDtypeStruct((N, HC + 2), f32),
        jax.ShapeDtypeStruct((16, 1), f32),
    ),
)


def _fin_body(acc_ref, den_ref, prev_ref, bias_ref, g_ref, b_ref, w1_ref,
              b1_ref, out_ref):
    acc = acc_ref[...]
    den = jnp.sum(den_ref[...], axis=0)
    out = acc / (den[:, None] + 1e-16) + bias_ref[...] + prev_ref[...]
    m = jnp.mean(out)
    v = jnp.mean((out - m) ** 2)
    out = (out - m) / jnp.sqrt(v + 1e-5) * g_ref[...] + b_ref[...]
    out = out * (1.0 / (1.0 + jnp.exp(-out)))
    x3 = jnp.mean(out, axis=0, keepdims=True)
    xo = jnp.dot(x3, w1_ref[...], preferred_element_type=f32) + b1_ref[...]
    out_ref[...] = xo * (1.0 / (1.0 + jnp.exp(-xo)))


_fin = pl.pallas_call(
    _fin_body,
    out_shape=jax.ShapeDtypeStruct((1, HC2), f32),
)


def _sc_edge_body(xpf_hbm, dl_hbm, dr_hbm, de_hbm, src_hbm, dst_hbm, et_hbm,
                  zacc_hbm, zden_hbm, acc_out, den_out,
                  dl_v, dr_v, de_v, srcv, dstv, exv, denv,
                  sidx0, sidx1, dchunk, rows0, rows1, acc_sh, sem0, sem1):
    c = lax.axis_index("c")
    s = lax.axis_index("s")
    base = s * EPT

    pltpu.sync_copy(dl_hbm, dl_v)
    pltpu.sync_copy(dr_hbm, dr_v)
    pltpu.sync_copy(de_hbm, de_v)
    pltpu.sync_copy(src_hbm.at[pl.ds(base, EPT)], srcv)
    pltpu.sync_copy(dst_hbm.at[pl.ds(base, EPT)], dstv)
    pltpu.sync_copy(zden_hbm, denv)

    def p1c(jc, carry):
        pltpu.sync_copy(et_hbm.at[pl.ds(base + jc * CH, CH)], dchunk)

        @plsc.parallel_loop(0, CH // 16, unroll=4)
        def p1(i):
            dv = dstv[pl.ds(jc * CH + i * 16, 16)]
            sv = srcv[pl.ds(jc * CH + i * 16, 16)]
            tv = dchunk[pl.ds(i * 16, 16)]
            a = (plsc.load_gather(dl_v, [dv]) + plsc.load_gather(dr_v, [sv])
                 + plsc.load_gather(de_v, [tv]))
            ex = jnp.exp(a * (1.0 / (1.0 + jnp.exp(-a))))
            exv[pl.ds(jc * CH + i * 16, 16)] = ex
            plsc.addupdate_scatter(denv, [dv], ex)
        return carry

    lax.fori_loop(0, EPT // CH, p1c, 0)

    @pl.when(c == 0)
    def _emit_den():
        pltpu.sync_copy(denv, den_out.at[s])

    def _mkidx(buf, j, add_off):
        @plsc.parallel_loop(0, CH // 16, unroll=4)
        def mk(r):
            buf[pl.ds(r * 16, 16)] = srcv[pl.ds(j * CH + r * 16, 16)] + add_off

    def _mkdst(j):
        @plsc.parallel_loop(0, CH // 16, unroll=4)
        def mk(r):
            dchunk[pl.ds(r * 16, 16)] = dstv[pl.ds(j * CH + r * 16, 16)]

    sidx = (sidx0, sidx1)
    rows = (rows0, rows1)
    sems = (sem0, sem1)
    for h in range(2):
        q = 2 * c + h
        coff = q * N

        @pl.when(s == 0)
        def _zero_acc():
            pltpu.sync_copy(zacc_hbm, acc_sh)

        plsc.subcore_barrier()

        for b in range(2):
            _mkidx(sidx[b], b, coff)
            pltpu.async_copy(xpf_hbm.at[sidx[b]], rows[b], sems[b])

        def grp(g, carry):
            for b in range(2):
                j = g * 2 + b
                pltpu.make_async_copy(xpf_hbm.at[sidx[b]], rows[b],
                                      sems[b]).wait()

                @plsc.parallel_loop(0, CH, unroll=8)
                def scale(r, _b=b, _j=j):
                    e = plsc.load_gather(
                        exv, [jnp.full((16,), _j * CH, i32) + r])
                    rows[_b][r, pl.ds(0, 16)] = rows[_b][r, pl.ds(0, 16)] * e
                _mkdst(j)
                pltpu.sync_copy(rows[b], acc_sh.at[dchunk], add=True)

                @pl.when(j + 2 < NCHUNK)
                def _prefetch(_b=b, _j=j):
                    _mkidx(sidx[_b], _j + 2, coff)
                    pltpu.async_copy(xpf_hbm.at[sidx[_b]], rows[_b], sems[_b])

            return carry

        lax.fori_loop(0, NCHUNK // 2, grp, 0)
        plsc.subcore_barrier()

        @pl.when(s == 0)
        def _emit_acc():
            pltpu.sync_copy(acc_sh, acc_out.at[q])


_sc_edge = functools.partial(
    pl.kernel,
    out_type=[
        jax.ShapeDtypeStruct((4, N, HQ), f32),
        jax.ShapeDtypeStruct((NS, N), f32),
    ],
    mesh=plsc.VectorSubcoreMesh(core_axis_name="c", subcore_axis_name="s"),
    compiler_params=pltpu.CompilerParams(needs_layout_passes=False,
                                         use_tc_tiling_on_sc=False),
    scratch_types=[
        pltpu.VMEM((N,), f32),
        pltpu.VMEM((N,), f32),
        pltpu.VMEM((16,), f32),
        pltpu.VMEM((EPT,), i32),
        pltpu.VMEM((EPT,), i32),
        pltpu.VMEM((EPT,), f32),
        pltpu.VMEM((N,), f32),
        pltpu.VMEM((CH,), i32),
        pltpu.VMEM((CH,), i32),
        pltpu.VMEM((CH,), i32),
        pltpu.VMEM((CH, HQ), f32),
        pltpu.VMEM((CH, HQ), f32),
        pltpu.VMEM_SHARED((N, HQ), f32),
        pltpu.SemaphoreType.DMA,
        pltpu.SemaphoreType.DMA,
    ],
)(_sc_edge_body)


def kernel(data_batch_1, data_batch_2, edge_index_1, edge_index_2,
           edge_type_1, edge_type_2, W0, W_rest, rel_emb,
           attn_l, attn_r, attn_e, conv_bias, ln_gamma, ln_beta,
           ln1_gamma, ln1_beta, lin1_W, lin1_b):
    pos = data_batch_1[:, :3]
    feat = data_batch_1[:, 3:]
    src = edge_index_1[0]
    dst = edge_index_1[1]
    et = edge_type_1

    posn, cov, featn = _pre(pos, feat, ln1_gamma.reshape(1, D - 3),
                            ln1_beta.reshape(1, D - 3))
    ev, V = jnp.linalg.eigh(cov)
    V = V[:, jnp.argsort(-ev)]

    rel_pad = jnp.concatenate([rel_emb, jnp.zeros((L, 1, HC), f32)], axis=1)
    zacc = jnp.zeros((N, HQ), f32)
    zden = jnp.zeros((N,), f32)

    xpe, de = _d0(posn, V, featn, W0,
                  attn_l[0].reshape(HC, 1), attn_r[0].reshape(HC, 1),
                  rel_pad[0], attn_e[0].reshape(1, HC))

    prev = jnp.zeros((N, HC), f32)
    for i in range(L):
        xpf = xpe[:, :HC].reshape(N, 4, HQ).transpose(1, 0, 2).reshape(4 * N, HQ)
        acc4, den16 = _sc_edge(xpf, xpe[:, HC], xpe[:, HC + 1],
                               de.reshape(16), src, dst, et, zacc, zden)
        acc = acc4.transpose(1, 0, 2).reshape(N, HC)
        if i < L - 1:
            prev, xpe, de = _mid(
                acc, den16, prev, conv_bias[i].reshape(1, HC),
                ln_gamma[i].reshape(1, HC), ln_beta[i].reshape(1, HC),
                W_rest[i], attn_l[i + 1].reshape(HC, 1),
                attn_r[i + 1].reshape(HC, 1), rel_pad[i + 1],
                attn_e[i + 1].reshape(1, HC))
        else:
            xo = _fin(acc, den16, prev, conv_bias[i].reshape(1, HC),
                      ln_gamma[i].reshape(1, HC), ln_beta[i].reshape(1, HC),
                      lin1_W, lin1_b.reshape(1, HC2))
    return (xo, xo)


# speedup vs baseline: 25.3411x; 1.0806x over previous
import functools

import jax
import jax.numpy as jnp
from jax import lax
from jax.experimental import pallas as pl
from jax.experimental.pallas import tpu as pltpu
from jax.experimental.pallas import tpu_sc as plsc

N = 10000
E = 320000
D = 128
HC = 64
HQ = HC // 4
L = 4
R = 15
HC2 = 256

NC = 2
NS = 16
EPT = E // NS
CH = 400
NCHUNK = EPT // CH

f32 = jnp.float32
i32 = jnp.int32


def _pre_body(pos_ref, feat_ref, g_ref, b_ref, posn_ref, cov_ref, featn_ref):
    pos = pos_ref[...]
    pc = pos - jnp.mean(pos, axis=0, keepdims=True)
    posn = pc * ((1.0 / jnp.max(jnp.abs(pc))) * 0.999999)
    posn_ref[...] = posn
    p = posn - jnp.mean(posn, axis=0, keepdims=True)
    cov_ref[...] = jnp.sum(p[:, :, None] * p[:, None, :], axis=0)
    f = feat_ref[...]
    fm = jnp.mean(f)
    fv = jnp.mean((f - fm) ** 2)
    featn_ref[...] = (f - fm) / jnp.sqrt(fv + 1e-5) * g_ref[...] + b_ref[...]


_pre = pl.pallas_call(
    _pre_body,
    out_shape=(
        jax.ShapeDtypeStruct((N, 3), f32),
        jax.ShapeDtypeStruct((3, 3), f32),
        jax.ShapeDtypeStruct((N, D - 3), f32),
    ),
)


def _d0_body(posn_ref, v_ref, featn_ref, w_ref, al_ref, ar_ref, rel_ref,
             ae_ref, xpe_ref, de_ref):
    x = jnp.concatenate(
        [jnp.dot(posn_ref[...], v_ref[...], preferred_element_type=f32),
         featn_ref[...]], axis=1)
    xp = jnp.dot(x, w_ref[...], preferred_element_type=f32)
    dl = jnp.dot(xp, al_ref[...], preferred_element_type=f32)
    dr = jnp.dot(xp, ar_ref[...], preferred_element_type=f32)
    xpe_ref[...] = jnp.concatenate([xp, dl, dr], axis=1)
    de_ref[...] = jnp.sum(rel_ref[...] * ae_ref[...], axis=1, keepdims=True)


_d0 = pl.pallas_call(
    _d0_body,
    out_shape=(
        jax.ShapeDtypeStruct((N, HC + 2), f32),
        jax.ShapeDtypeStruct((16, 1), f32),
    ),
)


def _mid_body(acc_ref, den_ref, prev_ref, bias_ref, g_ref, b_ref, w_ref,
              al_ref, ar_ref, rel_ref, ae_ref,
              out_ref, xpe_ref, de_ref):
    acc = acc_ref[...]
    den = jnp.sum(den_ref[...], axis=0)
    out = acc / (den[:, None] + 1e-16) + bias_ref[...] + prev_ref[...]
    m = jnp.mean(out)
    v = jnp.mean((out - m) ** 2)
    out = (out - m) / jnp.sqrt(v + 1e-5) * g_ref[...] + b_ref[...]
    out = out * (1.0 / (1.0 + jnp.exp(-out)))
    out_ref[...] = out
    xp = jnp.dot(out, w_ref[...], preferred_element_type=f32)
    dl = jnp.dot(xp, al_ref[...], preferred_element_type=f32)
    dr = jnp.dot(xp, ar_ref[...], preferred_element_type=f32)
    xpe_ref[...] = jnp.concatenate([xp, dl, dr], axis=1)
    de_ref[...] = jnp.sum(rel_ref[...] * ae_ref[...], axis=1, keepdims=True)


_mid = pl.pallas_call(
    _mid_body,
    out_shape=(
        jax.ShapeDtypeStruct((N, HC), f32),
        jax.ShapeDtypeStruct((N, HC + 2), f32),
        jax.ShapeDtypeStruct((16, 1), f32),
    ),
)


def _fin_body(acc_ref, den_ref, prev_ref, bias_ref, g_ref, b_ref, w1_ref,
              b1_ref, out_ref):
    acc = acc_ref[...]
    den = jnp.sum(den_ref[...], axis=0)
    out = acc / (den[:, None] + 1e-16) + bias_ref[...] + prev_ref[...]
    m = jnp.mean(out)
    v = jnp.mean((out - m) ** 2)
    out = (out - m) / jnp.sqrt(v + 1e-5) * g_ref[...] + b_ref[...]
    out = out * (1.0 / (1.0 + jnp.exp(-out)))
    x3 = jnp.mean(out, axis=0, keepdims=True)
    xo = jnp.dot(x3, w1_ref[...], preferred_element_type=f32) + b1_ref[...]
    out_ref[...] = xo * (1.0 / (1.0 + jnp.exp(-xo)))


_fin = pl.pallas_call(
    _fin_body,
    out_shape=jax.ShapeDtypeStruct((1, HC2), f32),
)


def _sc_edge_body(xpf_hbm, dl_hbm, dr_hbm, de_hbm, src_hbm, dst_hbm, et_hbm,
                  zacc_hbm, zden_hbm, acc_out, den_out,
                  dl_v, dr_v, de_v, srcv, dstv, exv, denv,
                  sidx0, sidx1, dchunk, rows0, rows1, acc_sh, sem0, sem1):
    c = lax.axis_index("c")
    s = lax.axis_index("s")
    base = s * EPT

    stg = [pltpu.async_copy(dl_hbm, dl_v, sem0),
           pltpu.async_copy(dr_hbm, dr_v, sem0),
           pltpu.async_copy(de_hbm, de_v, sem0),
           pltpu.async_copy(src_hbm.at[pl.ds(base, EPT)], srcv, sem1),
           pltpu.async_copy(dst_hbm.at[pl.ds(base, EPT)], dstv, sem1),
           pltpu.async_copy(zden_hbm, denv, sem1)]
    for cp in stg:
        cp.wait()

    etb = (sidx0, sidx1)
    sems = (sem0, sem1)
    for b in range(2):
        pltpu.async_copy(et_hbm.at[pl.ds(base + b * CH, CH)], etb[b], sems[b])

    def p1c(g, carry):
        for b in range(2):
            jc = g * 2 + b
            pltpu.make_async_copy(et_hbm.at[pl.ds(base + jc * CH, CH)],
                                  etb[b], sems[b]).wait()

            @plsc.parallel_loop(0, CH // 16, unroll=4)
            def p1(i, _b=b, _jc=jc):
                dv = dstv[pl.ds(_jc * CH + i * 16, 16)]
                sv = srcv[pl.ds(_jc * CH + i * 16, 16)]
                tv = etb[_b][pl.ds(i * 16, 16)]
                a = (plsc.load_gather(dl_v, [dv])
                     + plsc.load_gather(dr_v, [sv])
                     + plsc.load_gather(de_v, [tv]))
                ex = jnp.exp(a * (1.0 / (1.0 + jnp.exp(-a))))
                exv[pl.ds(_jc * CH + i * 16, 16)] = ex
                plsc.addupdate_scatter(denv, [dv], ex)

            @pl.when(jc + 2 < NCHUNK)
            def _pf1(_b=b, _jc=jc):
                pltpu.async_copy(et_hbm.at[pl.ds(base + (_jc + 2) * CH, CH)],
                                 etb[_b], sems[_b])
        return carry

    lax.fori_loop(0, NCHUNK // 2, p1c, 0)

    @pl.when(c == 0)
    def _emit_den():
        pltpu.sync_copy(denv, den_out.at[s])

    def _mkidx(buf, j, add_off):
        @plsc.parallel_loop(0, CH // 16, unroll=4)
        def mk(r):
            buf[pl.ds(r * 16, 16)] = srcv[pl.ds(j * CH + r * 16, 16)] + add_off

    def _mkdst(j):
        @plsc.parallel_loop(0, CH // 16, unroll=4)
        def mk(r):
            dchunk[pl.ds(r * 16, 16)] = dstv[pl.ds(j * CH + r * 16, 16)]

    sidx = etb
    rows = (rows0, rows1)
    for h in range(2):
        q = 2 * c + h
        coff = q * N

        @pl.when(s == 0)
        def _zero_acc():
            pltpu.sync_copy(zacc_hbm, acc_sh)

        plsc.subcore_barrier()

        for b in range(2):
            _mkidx(sidx[b], b, coff)
            pltpu.async_copy(xpf_hbm.at[sidx[b]], rows[b], sems[b])

        def grp(g, carry):
            for b in range(2):
                j = g * 2 + b
                pltpu.make_async_copy(xpf_hbm.at[sidx[b]], rows[b],
                                      sems[b]).wait()

                @plsc.parallel_loop(0, CH, unroll=8)
                def scale(r, _b=b, _j=j):
                    e = plsc.load_gather(
                        exv, [jnp.full((16,), _j * CH, i32) + r])
                    rows[_b][r, pl.ds(0, 16)] = rows[_b][r, pl.ds(0, 16)] * e
                _mkdst(j)
                pltpu.sync_copy(rows[b], acc_sh.at[dchunk], add=True)

                @pl.when(j + 2 < NCHUNK)
                def _prefetch(_b=b, _j=j):
                    _mkidx(sidx[_b], _j + 2, coff)
                    pltpu.async_copy(xpf_hbm.at[sidx[_b]], rows[_b], sems[_b])

            return carry

        lax.fori_loop(0, NCHUNK // 2, grp, 0)
        plsc.subcore_barrier()

        @pl.when(s == 0)
        def _emit_acc():
            pltpu.sync_copy(acc_sh, acc_out.at[q])


_sc_edge = functools.partial(
    pl.kernel,
    out_type=[
        jax.ShapeDtypeStruct((4, N, HQ), f32),
        jax.ShapeDtypeStruct((NS, N), f32),
    ],
    mesh=plsc.VectorSubcoreMesh(core_axis_name="c", subcore_axis_name="s"),
    compiler_params=pltpu.CompilerParams(needs_layout_passes=False,
                                         use_tc_tiling_on_sc=False),
    scratch_types=[
        pltpu.VMEM((N,), f32),
        pltpu.VMEM((N,), f32),
        pltpu.VMEM((16,), f32),
        pltpu.VMEM((EPT,), i32),
        pltpu.VMEM((EPT,), i32),
        pltpu.VMEM((EPT,), f32),
        pltpu.VMEM((N,), f32),
        pltpu.VMEM((CH,), i32),
        pltpu.VMEM((CH,), i32),
        pltpu.VMEM((CH,), i32),
        pltpu.VMEM((CH, HQ), f32),
        pltpu.VMEM((CH, HQ), f32),
        pltpu.VMEM_SHARED((N, HQ), f32),
        pltpu.SemaphoreType.DMA,
        pltpu.SemaphoreType.DMA,
    ],
)(_sc_edge_body)


def kernel(data_batch_1, data_batch_2, edge_index_1, edge_index_2,
           edge_type_1, edge_type_2, W0, W_rest, rel_emb,
           attn_l, attn_r, attn_e, conv_bias, ln_gamma, ln_beta,
           ln1_gamma, ln1_beta, lin1_W, lin1_b):
    pos = data_batch_1[:, :3]
    feat = data_batch_1[:, 3:]
    src = edge_index_1[0]
    dst = edge_index_1[1]
    et = edge_type_1

    posn, cov, featn = _pre(pos, feat, ln1_gamma.reshape(1, D - 3),
                            ln1_beta.reshape(1, D - 3))
    ev, V = jnp.linalg.eigh(cov)
    V = V[:, jnp.argsort(-ev)]

    rel_pad = jnp.concatenate([rel_emb, jnp.zeros((L, 1, HC), f32)], axis=1)
    zacc = jnp.zeros((N, HQ), f32)
    zden = jnp.zeros((N,), f32)

    xpe, de = _d0(posn, V, featn, W0,
                  attn_l[0].reshape(HC, 1), attn_r[0].reshape(HC, 1),
                  rel_pad[0], attn_e[0].reshape(1, HC))

    prev = jnp.zeros((N, HC), f32)
    for i in range(L):
        xpf = xpe[:, :HC].reshape(N, 4, HQ).transpose(1, 0, 2).reshape(4 * N, HQ)
        acc4, den16 = _sc_edge(xpf, xpe[:, HC], xpe[:, HC + 1],
                               de.reshape(16), src, dst, et, zacc, zden)
        acc = acc4.transpose(1, 0, 2).reshape(N, HC)
        if i < L - 1:
            prev, xpe, de = _mid(
                acc, den16, prev, conv_bias[i].reshape(1, HC),
                ln_gamma[i].reshape(1, HC), ln_beta[i].reshape(1, HC),
                W_rest[i], attn_l[i + 1].reshape(HC, 1),
                attn_r[i + 1].reshape(HC, 1), rel_pad[i + 1],
                attn_e[i + 1].reshape(1, HC))
        else:
            xo = _fin(acc, den16, prev, conv_bias[i].reshape(1, HC),
                      ln_gamma[i].reshape(1, HC), ln_beta[i].reshape(1, HC),
                      lin1_W, lin1_b.reshape(1, HC2))
    return (xo, xo)


# speedup vs baseline: 25.4278x; 1.0034x over previous
import functools

import jax
import jax.numpy as jnp
from jax import lax
from jax.experimental import pallas as pl
from jax.experimental.pallas import tpu as pltpu
from jax.experimental.pallas import tpu_sc as plsc

N = 10000
E = 320000
D = 128
HC = 64
HQ = HC // 4
L = 4
R = 15
HC2 = 256

NC = 2
NS = 16
EPT = E // NS
CH = 400
NCHUNK = EPT // CH

f32 = jnp.float32
i32 = jnp.int32


def _pre_body(pos_ref, feat_ref, g_ref, b_ref, posn_ref, cov_ref, featn_ref):
    pos = pos_ref[...]
    pc = pos - jnp.mean(pos, axis=0, keepdims=True)
    posn = pc * ((1.0 / jnp.max(jnp.abs(pc))) * 0.999999)
    posn_ref[...] = posn
    p = posn - jnp.mean(posn, axis=0, keepdims=True)
    cov_ref[...] = jnp.sum(p[:, :, None] * p[:, None, :], axis=0)
    f = feat_ref[...]
    fm = jnp.mean(f)
    fv = jnp.mean((f - fm) ** 2)
    featn_ref[...] = (f - fm) / jnp.sqrt(fv + 1e-5) * g_ref[...] + b_ref[...]


_pre = pl.pallas_call(
    _pre_body,
    out_shape=(
        jax.ShapeDtypeStruct((N, 3), f32),
        jax.ShapeDtypeStruct((3, 3), f32),
        jax.ShapeDtypeStruct((N, D - 3), f32),
    ),
)


def _d0_body(posn_ref, v_ref, featn_ref, w_ref, al_ref, ar_ref, rel_ref,
             ae_ref, xpe_ref, de_ref):
    x = jnp.concatenate(
        [jnp.dot(posn_ref[...], v_ref[...], preferred_element_type=f32),
         featn_ref[...]], axis=1)
    xp = jnp.dot(x, w_ref[...], preferred_element_type=f32)
    dl = jnp.dot(xp, al_ref[...], preferred_element_type=f32)
    dr = jnp.dot(xp, ar_ref[...], preferred_element_type=f32)
    xpe_ref[...] = jnp.concatenate([xp, dl, dr], axis=1)
    de_ref[...] = jnp.sum(rel_ref[...] * ae_ref[...], axis=1, keepdims=True)


_d0 = pl.pallas_call(
    _d0_body,
    out_shape=(
        jax.ShapeDtypeStruct((N, HC + 2), f32),
        jax.ShapeDtypeStruct((16, 1), f32),
    ),
)


def _mid_body(acc_ref, den_ref, prev_ref, bias_ref, g_ref, b_ref, w_ref,
              al_ref, ar_ref, rel_ref, ae_ref,
              out_ref, xpe_ref, de_ref):
    acc = acc_ref[...]
    den = jnp.sum(den_ref[...], axis=0)
    out = acc / (den[:, None] + 1e-16) + bias_ref[...] + prev_ref[...]
    m = jnp.mean(out)
    v = jnp.mean((out - m) ** 2)
    out = (out - m) / jnp.sqrt(v + 1e-5) * g_ref[...] + b_ref[...]
    out = out * (1.0 / (1.0 + jnp.exp(-out)))
    out_ref[...] = out
    xp = jnp.dot(out, w_ref[...], preferred_element_type=f32)
    dl = jnp.dot(xp, al_ref[...], preferred_element_type=f32)
    dr = jnp.dot(xp, ar_ref[...], preferred_element_type=f32)
    xpe_ref[...] = jnp.concatenate([xp, dl, dr], axis=1)
    de_ref[...] = jnp.sum(rel_ref[...] * ae_ref[...], axis=1, keepdims=True)


_mid = pl.pallas_call(
    _mid_body,
    out_shape=(
        jax.ShapeDtypeStruct((N, HC), f32),
        jax.ShapeDtypeStruct((N, HC + 2), f32),
        jax.ShapeDtypeStruct((16, 1), f32),
    ),
)


def _fin_body(acc_ref, den_ref, prev_ref, bias_ref, g_ref, b_ref, w1_ref,
              b1_ref, out_ref):
    acc = acc_ref[...]
    den = jnp.sum(den_ref[...], axis=0)
    out = acc / (den[:, None] + 1e-16) + bias_ref[...] + prev_ref[...]
    m = jnp.mean(out)
    v = jnp.mean((out - m) ** 2)
    out = (out - m) / jnp.sqrt(v + 1e-5) * g_ref[...] + b_ref[...]
    out = out * (1.0 / (1.0 + jnp.exp(-out)))
    x3 = jnp.mean(out, axis=0, keepdims=True)
    xo = jnp.dot(x3, w1_ref[...], preferred_element_type=f32) + b1_ref[...]
    out_ref[...] = xo * (1.0 / (1.0 + jnp.exp(-xo)))


_fin = pl.pallas_call(
    _fin_body,
    out_shape=jax.ShapeDtypeStruct((1, HC2), f32),
)


def _sc_edge_body(xpf_hbm, dl_hbm, dr_hbm, de_hbm, src_hbm, dst_hbm, et_hbm,
                  zacc_hbm, zden_hbm, acc_out, den_out,
                  dl_v, dr_v, de_v, srcv, dstv, exv, denv,
                  sidx0, sidx1, dchunk, rows0, rows1, acc_sh, sem0, sem1):
    c = lax.axis_index("c")
    s = lax.axis_index("s")
    base = s * EPT

    stg = [pltpu.async_copy(dl_hbm, dl_v, sem0),
           pltpu.async_copy(dr_hbm, dr_v, sem0),
           pltpu.async_copy(de_hbm, de_v, sem0),
           pltpu.async_copy(src_hbm.at[pl.ds(base, EPT)], srcv, sem1),
           pltpu.async_copy(dst_hbm.at[pl.ds(base, EPT)], dstv, sem1),
           pltpu.async_copy(zden_hbm, denv, sem1)]
    for cp in stg:
        cp.wait()

    etb = (sidx0, sidx1)
    sems = (sem0, sem1)
    for b in range(2):
        pltpu.async_copy(et_hbm.at[pl.ds(base + b * CH, CH)], etb[b], sems[b])

    def p1c(g, carry):
        for b in range(2):
            jc = g * 2 + b
            pltpu.make_async_copy(et_hbm.at[pl.ds(base + jc * CH, CH)],
                                  etb[b], sems[b]).wait()

            @plsc.parallel_loop(0, CH // 16, unroll=4)
            def p1(i, _b=b, _jc=jc):
                dv = dstv[pl.ds(_jc * CH + i * 16, 16)]
                sv = srcv[pl.ds(_jc * CH + i * 16, 16)]
                tv = etb[_b][pl.ds(i * 16, 16)]
                a = (plsc.load_gather(dl_v, [dv])
                     + plsc.load_gather(dr_v, [sv])
                     + plsc.load_gather(de_v, [tv]))
                ex = jnp.exp(a * (1.0 / (1.0 + jnp.exp(-a))))
                exv[pl.ds(_jc * CH + i * 16, 16)] = ex
                plsc.addupdate_scatter(denv, [dv], ex)

            @pl.when(jc + 2 < NCHUNK)
            def _pf1(_b=b, _jc=jc):
                pltpu.async_copy(et_hbm.at[pl.ds(base + (_jc + 2) * CH, CH)],
                                 etb[_b], sems[_b])
        return carry

    lax.fori_loop(0, NCHUNK // 2, p1c, 0)

    @pl.when(c == 0)
    def _emit_den():
        pltpu.sync_copy(denv, den_out.at[s])

    def _mkidx(buf, j, add_off):
        @plsc.parallel_loop(0, CH // 16, unroll=4)
        def mk(r):
            buf[pl.ds(r * 16, 16)] = srcv[pl.ds(j * CH + r * 16, 16)] + add_off

    def _mkdst(j):
        @plsc.parallel_loop(0, CH // 16, unroll=4)
        def mk(r):
            dchunk[pl.ds(r * 16, 16)] = dstv[pl.ds(j * CH + r * 16, 16)]

    sidx = etb
    rows = (rows0, rows1)
    for h in range(2):
        q = 2 * c + h
        coff = q * N

        @pl.when(s == 0)
        def _zero_acc():
            pltpu.sync_copy(zacc_hbm, acc_sh)

        plsc.subcore_barrier()

        for b in range(2):
            _mkidx(sidx[b], b, coff)
            pltpu.async_copy(xpf_hbm.at[sidx[b]], rows[b], sems[b])

        def grp(g, carry):
            for b in range(2):
                j = g * 2 + b
                pltpu.make_async_copy(xpf_hbm.at[sidx[b]], rows[b],
                                      sems[b]).wait()

                @plsc.parallel_loop(0, CH, unroll=16)
                def scale(r, _b=b, _j=j):
                    e = plsc.load_gather(
                        exv, [jnp.full((16,), _j * CH, i32) + r])
                    rows[_b][r, pl.ds(0, 16)] = rows[_b][r, pl.ds(0, 16)] * e
                _mkdst(j)
                pltpu.sync_copy(rows[b], acc_sh.at[dchunk], add=True)

                @pl.when(j + 2 < NCHUNK)
                def _prefetch(_b=b, _j=j):
                    _mkidx(sidx[_b], _j + 2, coff)
                    pltpu.async_copy(xpf_hbm.at[sidx[_b]], rows[_b], sems[_b])

            return carry

        lax.fori_loop(0, NCHUNK // 2, grp, 0)
        plsc.subcore_barrier()

        @pl.when(s == 0)
        def _emit_acc():
            pltpu.sync_copy(acc_sh, acc_out.at[q])


_sc_edge = functools.partial(
    pl.kernel,
    out_type=[
        jax.ShapeDtypeStruct((4, N, HQ), f32),
        jax.ShapeDtypeStruct((NS, N), f32),
    ],
    mesh=plsc.VectorSubcoreMesh(core_axis_name="c", subcore_axis_name="s"),
    compiler_params=pltpu.CompilerParams(needs_layout_passes=False,
                                         use_tc_tiling_on_sc=False),
    scratch_types=[
        pltpu.VMEM((N,), f32),
        pltpu.VMEM((N,), f32),
        pltpu.VMEM((16,), f32),
        pltpu.VMEM((EPT,), i32),
        pltpu.VMEM((EPT,), i32),
        pltpu.VMEM((EPT,), f32),
        pltpu.VMEM((N,), f32),
        pltpu.VMEM((CH,), i32),
        pltpu.VMEM((CH,), i32),
        pltpu.VMEM((CH,), i32),
        pltpu.VMEM((CH, HQ), f32),
        pltpu.VMEM((CH, HQ), f32),
        pltpu.VMEM_SHARED((N, HQ), f32),
        pltpu.SemaphoreType.DMA,
        pltpu.SemaphoreType.DMA,
    ],
)(_sc_edge_body)


def kernel(data_batch_1, data_batch_2, edge_index_1, edge_index_2,
           edge_type_1, edge_type_2, W0, W_rest, rel_emb,
           attn_l, attn_r, attn_e, conv_bias, ln_gamma, ln_beta,
           ln1_gamma, ln1_beta, lin1_W, lin1_b):
    pos = data_batch_1[:, :3]
    feat = data_batch_1[:, 3:]
    src = edge_index_1[0]
    dst = edge_index_1[1]
    et = edge_type_1

    posn, cov, featn = _pre(pos, feat, ln1_gamma.reshape(1, D - 3),
                            ln1_beta.reshape(1, D - 3))
    ev, V = jnp.linalg.eigh(cov)
    V = V[:, jnp.argsort(-ev)]

    rel_pad = jnp.concatenate([rel_emb, jnp.zeros((L, 1, HC), f32)], axis=1)
    zacc = jnp.zeros((N, HQ), f32)
    zden = jnp.zeros((N,), f32)

    xpe, de = _d0(posn, V, featn, W0,
                  attn_l[0].reshape(HC, 1), attn_r[0].reshape(HC, 1),
                  rel_pad[0], attn_e[0].reshape(1, HC))

    prev = jnp.zeros((N, HC), f32)
    for i in range(L):
        xpf = xpe[:, :HC].reshape(N, 4, HQ).transpose(1, 0, 2).reshape(4 * N, HQ)
        acc4, den16 = _sc_edge(xpf, xpe[:, HC], xpe[:, HC + 1],
                               de.reshape(16), src, dst, et, zacc, zden)
        acc = acc4.transpose(1, 0, 2).reshape(N, HC)
        if i < L - 1:
            prev, xpe, de = _mid(
                acc, den16, prev, conv_bias[i].reshape(1, HC),
                ln_gamma[i].reshape(1, HC), ln_beta[i].reshape(1, HC),
                W_rest[i], attn_l[i + 1].reshape(HC, 1),
                attn_r[i + 1].reshape(HC, 1), rel_pad[i + 1],
                attn_e[i + 1].reshape(1, HC))
        else:
            xo = _fin(acc, den16, prev, conv_bias[i].reshape(1, HC),
                      ln_gamma[i].reshape(1, HC), ln_beta[i].reshape(1, HC),
                      lin1_W, lin1_b.reshape(1, HC2))
    return (xo, xo)
